# baseline (speedup 1.0000x reference)
import functools

import jax
import jax.numpy as jnp
import numpy as np
from jax import lax
from jax.experimental import pallas as pl
from jax.experimental.pallas import tpu as pltpu
from jax.experimental.pallas import tpu_sc as plsc

B = 8
FW = 20000
FWP = 20096
NBLK = 157
NBLK_P = 160
BLK = 128
MAX_OUT = 50
NEG = np.float32(-1e30)
NEG_HALF = np.float32(-5e29)
DIST = np.float32(16.0)
THR = np.float32(0.7)
LOG2E = np.float32(1.4426950408889634)
SEL_PAD = np.float32(3e38)


KM = NBLK
KP0 = NBLK
KP1 = NBLK
KBM = 2
KTOT = 3 * NBLK + KBM


def _prep_body(logit_ref, d0_ref, d1_ref, wl_ref, out_ref):
    x = logit_ref[...]
    e = jnp.exp2(x * (-LOG2E))
    s = 1.0 / (1.0 + e)
    iot = lax.broadcasted_iota(jnp.int32, (B, FW), 1).astype(jnp.float32)
    ic = (iot + 0.5) * 16.0
    p0 = d0_ref[...] * 16.0 + ic
    p1 = d1_ref[...] * 16.0 + ic
    wl = wl_ref[0]
    valid = (p0 >= 0.0) & (p0 <= wl) & (p1 >= 0.0) & (p1 <= wl) & (s >= THR)
    mm = jnp.where(valid, s, NEG)
    negpad = jnp.full((B, FWP - FW), NEG, jnp.float32)
    zpad = jnp.zeros((B, FWP - FW), jnp.float32)
    mp = jnp.concatenate([mm, negpad], axis=1)
    p0p = jnp.concatenate([p0, zpad], axis=1)
    p1p = jnp.concatenate([p1, zpad], axis=1)
    bm = jnp.max(mp.reshape(B, NBLK, BLK), axis=2)
    bmp = jnp.concatenate(
        [bm, jnp.full((B, 256 - NBLK), NEG, jnp.float32)], axis=1)
    for k in range(NBLK):
        out_ref[k] = mp[:, k * BLK:(k + 1) * BLK]
    for k in range(NBLK):
        out_ref[KM + k] = p0p[:, k * BLK:(k + 1) * BLK]
    for k in range(NBLK):
        out_ref[KM + KP0 + k] = p1p[:, k * BLK:(k + 1) * BLK]
    out_ref[3 * NBLK] = bmp[:, :BLK]
    out_ref[3 * NBLK + 1] = bmp[:, BLK:]


def _st1(ref, idx, val):
    iota = lax.broadcasted_iota(jnp.int32, (16,), 0)
    idxv = jnp.zeros((16,), jnp.int32) + idx
    valv = jnp.zeros((16,), jnp.float32) + val
    plsc.store_scatter(ref, [idxv], valv, mask=iota == 0)


def _st1_2d(ref, r, c, val):
    iota = lax.broadcasted_iota(jnp.int32, (16,), 0)
    rv = jnp.zeros((16,), jnp.int32) + r
    cv = jnp.zeros((16,), jnp.int32) + c
    valv = jnp.zeros((16,), jnp.float32) + val
    plsc.store_scatter(ref, [rv, cv], valv, mask=iota == 0)


def _nms_walk_body(prep_hbm, out_hbm,
                   m_loc, p0_loc, p1_loc, bm_loc, selc_loc,
                   out_loc, sem):
    wid = lax.axis_index("s") * 2 + lax.axis_index("c")

    @pl.when(wid < B)
    def _run():
        b = wid
        cp_m = pltpu.make_async_copy(
            prep_hbm.at[pl.ds(0, NBLK), b], m_loc, sem)
        cp_p0 = pltpu.make_async_copy(
            prep_hbm.at[pl.ds(KM, NBLK), b], p0_loc, sem)
        cp_p1 = pltpu.make_async_copy(
            prep_hbm.at[pl.ds(2 * NBLK, NBLK), b], p1_loc, sem)
        cp_bm = pltpu.make_async_copy(
            prep_hbm.at[pl.ds(3 * NBLK, KBM), b], bm_loc, sem)
        cp_m.start()
        cp_p0.start()
        cp_p1.start()
        cp_bm.start()

        zero16 = jnp.zeros((16,), jnp.float32)
        for i in range(24):
            out_loc[pl.ds(i * 16, 16)] = zero16
        for i in range(4):
            selc_loc[pl.ds(i * 16, 16)] = jnp.full((16,), SEL_PAD, jnp.float32)

        cp_m.wait()
        cp_p0.wait()
        cp_p1.wait()
        cp_bm.wait()

        iota16 = lax.broadcasted_iota(jnp.int32, (16,), 0)
        big = jnp.int32(10**9)

        def _cond(carry):
            cnt, done = carry
            return (cnt < MAX_OUT) & jnp.logical_not(done)

        def _step(carry):
            cnt, _ = carry
            gmv = [bm_loc[(j * 16) // BLK, pl.ds((j * 16) % BLK, 16)]
                   for j in range(10)]
            gm = gmv[0]
            for j in range(1, 10):
                gm = jnp.maximum(gm, gmv[j])
            mval = jnp.max(gm)
            exhausted = mval <= NEG_HALF
            b_star = jnp.int32(big)
            for j in range(9, -1, -1):
                fj = plsc.all_reduce_ffs(gmv[j] == mval)
                if fj.ndim:
                    fj = fj[0]
                b_star = jnp.where((fj >= 0) & (fj < 16),
                                   j * 16 + fj, b_star)
            mv = [m_loc[b_star, pl.ds(j * 16, 16)] for j in range(8)]
            qc = jnp.int32(big)
            for j in range(7, -1, -1):
                fj = plsc.all_reduce_ffs(mv[j] == mval)
                if fj.ndim:
                    fj = fj[0]
                qc = jnp.where((fj >= 0) & (fj < 16), j * 16 + fj, qc)
            zeros16 = jnp.zeros((16,), jnp.int32)
            bsv = b_star + zeros16
            qcv = qc + zeros16
            p0v = plsc.load_gather(p0_loc, [bsv, qcv])
            p1v = plsc.load_gather(p1_loc, [bsv, qcv])
            cvec = (p0v + p1v) * 0.5
            near = jnp.abs(selc_loc[pl.ds(0, 16)] - cvec) <= DIST
            for j in range(1, 4):
                near = near | (
                    jnp.abs(selc_loc[pl.ds(j * 16, 16)] - cvec) <= DIST)
            nearcnt = plsc.all_reduce_population_count(near)
            if nearcnt.ndim:
                nearcnt = nearcnt[0]
            keep = (nearcnt == 0) & jnp.logical_not(exhausted)

            @pl.when(jnp.logical_not(exhausted))
            def _pop():
                _st1_2d(m_loc, b_star, qc, NEG)
                acc = jnp.full((16,), NEG, jnp.float32)
                for j in range(8):
                    acc = jnp.maximum(
                        acc, jnp.where(j * 16 + iota16 == qc, NEG, mv[j]))
                _st1_2d(bm_loc, b_star >> 7, b_star & 127, jnp.max(acc))

            @pl.when(keep)
            def _emit():
                iotav = jnp.zeros((16,), jnp.int32) + cnt
                plsc.store_scatter(selc_loc, [iotav], cvec, mask=iota16 == 0)
                posv = jnp.where(iota16 == 0, p0v,
                                 jnp.where(iota16 == 1, p1v, 1.0))
                plsc.store_scatter(out_loc, [4 * cnt + iota16], posv,
                                   mask=iota16 < 3)
                scv = jnp.where(iota16 == 0, mval, 1.0)
                plsc.store_scatter(out_loc, [256 + 2 * cnt + iota16], scv,
                                   mask=iota16 < 2)

            cnt_next = cnt + jnp.where(keep, 1, 0).astype(jnp.int32)
            return cnt_next, exhausted

        lax.while_loop(_cond, _step, (jnp.int32(0), jnp.bool_(False)))

        pltpu.sync_copy(out_loc,
                        out_hbm.at[pl.ds(pl.multiple_of(b * 384, 8), 384)])


def kernel(pred_cls_logit, pred_delta, img_width):
    wl = (jnp.float32(img_width) - 1.0).reshape(1)
    d0 = pred_delta[:, :, 0]
    d1 = pred_delta[:, :, 1]

    prep = pl.pallas_call(
        _prep_body,
        out_shape=jax.ShapeDtypeStruct((KTOT, B, BLK), jnp.float32),
        in_specs=[
            pl.BlockSpec(memory_space=pltpu.VMEM),
            pl.BlockSpec(memory_space=pltpu.VMEM),
            pl.BlockSpec(memory_space=pltpu.VMEM),
            pl.BlockSpec(memory_space=pltpu.SMEM),
        ],
        out_specs=pl.BlockSpec(memory_space=pltpu.VMEM),
    )(pred_cls_logit, d0, d1, wl)

    mesh = plsc.VectorSubcoreMesh(core_axis_name="c", subcore_axis_name="s",
                                  num_cores=2)
    out_flat = pl.kernel(
        _nms_walk_body,
        out_type=jax.ShapeDtypeStruct((B * 384,), jnp.float32),
        mesh=mesh,
        compiler_params=pltpu.CompilerParams(needs_layout_passes=False,
                                             skip_device_barrier=True),
        scratch_types=[
            pltpu.VMEM((NBLK, BLK), jnp.float32),
            pltpu.VMEM((NBLK, BLK), jnp.float32),
            pltpu.VMEM((NBLK, BLK), jnp.float32),
            pltpu.VMEM((KBM, BLK), jnp.float32),
            pltpu.VMEM((64,), jnp.float32),
            pltpu.VMEM((384,), jnp.float32),
            pltpu.SemaphoreType.DMA,
        ],
    )(prep)

    buf = out_flat.reshape(B, 384)
    nms_positions = buf[:, :256].reshape(B, 64, 4)[:, :MAX_OUT, :3]
    nms_scores = buf[:, 256:].reshape(B, 64, 2)[:, :MAX_OUT, :]
    return nms_positions, nms_scores
